# baseline (speedup 1.0000x reference)
import functools

import jax
import jax.numpy as jnp
from jax.experimental import pallas as pl

_N = 100000
_D = 128
_K = 32
_B = 4000


def _phase1_body(ev_ref, x_ref, eigval_ref, wf_ref, bf_ref, wo_ref, out_ref):
    i = pl.program_id(0)

    @pl.when(i == 0)
    def _init():
        out_ref[...] = jnp.zeros_like(out_ref)

    out_ref[...] += jax.lax.dot_general(
        ev_ref[...], x_ref[...],
        dimension_numbers=(((0,), (0,)), ((), ())),
        preferred_element_type=jnp.float32,
    )

    @pl.when(i == pl.num_programs(0) - 1)
    def _finish():
        spec = out_ref[...] * eigval_ref[...]
        gate = jax.nn.sigmoid(
            jnp.dot(spec, wf_ref[...], preferred_element_type=jnp.float32)
            + bf_ref[...]
        )
        spec = spec * gate
        out_ref[...] = jnp.dot(spec, wo_ref[...],
                               preferred_element_type=jnp.float32)


def _phase2_body(x_ref, ev_ref, spec2_ref, bo_ref, out_ref):
    out_ref[...] = (
        x_ref[...]
        + jnp.dot(ev_ref[...], spec2_ref[...],
                  preferred_element_type=jnp.float32)
        + bo_ref[...]
    )


@functools.partial(jax.jit, static_argnames=())
def kernel(x, eigvec, eigval, W_filter, b_filter, W_out, b_out):
    nblocks = _N // _B
    eigval2 = eigval.reshape(_K, 1)
    bf2 = b_filter.reshape(1, _D)
    bo2 = b_out.reshape(1, _D)

    spec2 = pl.pallas_call(
        _phase1_body,
        grid=(nblocks,),
        in_specs=[
            pl.BlockSpec((_B, _K), lambda i: (i, 0)),
            pl.BlockSpec((_B, _D), lambda i: (i, 0)),
            pl.BlockSpec((_K, 1), lambda i: (0, 0)),
            pl.BlockSpec((_D, _D), lambda i: (0, 0)),
            pl.BlockSpec((1, _D), lambda i: (0, 0)),
            pl.BlockSpec((_D, _D), lambda i: (0, 0)),
        ],
        out_specs=pl.BlockSpec((_K, _D), lambda i: (0, 0)),
        out_shape=jax.ShapeDtypeStruct((_K, _D), jnp.float32),
    )(eigvec, x, eigval2, W_filter, bf2, W_out)

    out = pl.pallas_call(
        _phase2_body,
        grid=(nblocks,),
        in_specs=[
            pl.BlockSpec((_B, _D), lambda i: (i, 0)),
            pl.BlockSpec((_B, _K), lambda i: (i, 0)),
            pl.BlockSpec((_K, _D), lambda i: (0, 0)),
            pl.BlockSpec((1, _D), lambda i: (0, 0)),
        ],
        out_specs=pl.BlockSpec((_B, _D), lambda i: (i, 0)),
        out_shape=jax.ShapeDtypeStruct((_N, _D), jnp.float32),
    )(x, eigvec, spec2, bo2)
    return out


# speedup vs baseline: 1.0601x; 1.0601x over previous
import functools

import jax
import jax.numpy as jnp
from jax.experimental import pallas as pl

_N = 100000
_D = 128
_K = 32
_B = 10000


def _phase1_body(ev_ref, x_ref, eigval_ref, wf_ref, bf_ref, wo_ref, out_ref):
    i = pl.program_id(0)

    @pl.when(i == 0)
    def _init():
        out_ref[...] = jnp.zeros_like(out_ref)

    out_ref[...] += jax.lax.dot_general(
        ev_ref[...], x_ref[...],
        dimension_numbers=(((0,), (0,)), ((), ())),
        preferred_element_type=jnp.float32,
    )

    @pl.when(i == pl.num_programs(0) - 1)
    def _finish():
        spec = out_ref[...] * eigval_ref[...]
        gate = jax.nn.sigmoid(
            jnp.dot(spec, wf_ref[...], preferred_element_type=jnp.float32)
            + bf_ref[...]
        )
        spec = spec * gate
        out_ref[...] = jnp.dot(spec, wo_ref[...],
                               preferred_element_type=jnp.float32)


def _phase2_body(x_ref, ev_ref, spec2_ref, bo_ref, out_ref):
    out_ref[...] = (
        x_ref[...]
        + jnp.dot(ev_ref[...], spec2_ref[...],
                  preferred_element_type=jnp.float32)
        + bo_ref[...]
    )


@functools.partial(jax.jit, static_argnames=())
def kernel(x, eigvec, eigval, W_filter, b_filter, W_out, b_out):
    nblocks = _N // _B
    eigval2 = eigval.reshape(_K, 1)
    bf2 = b_filter.reshape(1, _D)
    bo2 = b_out.reshape(1, _D)

    spec2 = pl.pallas_call(
        _phase1_body,
        grid=(nblocks,),
        in_specs=[
            pl.BlockSpec((_B, _K), lambda i: (i, 0)),
            pl.BlockSpec((_B, _D), lambda i: (i, 0)),
            pl.BlockSpec((_K, 1), lambda i: (0, 0)),
            pl.BlockSpec((_D, _D), lambda i: (0, 0)),
            pl.BlockSpec((1, _D), lambda i: (0, 0)),
            pl.BlockSpec((_D, _D), lambda i: (0, 0)),
        ],
        out_specs=pl.BlockSpec((_K, _D), lambda i: (0, 0)),
        out_shape=jax.ShapeDtypeStruct((_K, _D), jnp.float32),
    )(eigvec, x, eigval2, W_filter, bf2, W_out)

    out = pl.pallas_call(
        _phase2_body,
        grid=(nblocks,),
        in_specs=[
            pl.BlockSpec((_B, _D), lambda i: (i, 0)),
            pl.BlockSpec((_B, _K), lambda i: (i, 0)),
            pl.BlockSpec((_K, _D), lambda i: (0, 0)),
            pl.BlockSpec((1, _D), lambda i: (0, 0)),
        ],
        out_specs=pl.BlockSpec((_B, _D), lambda i: (i, 0)),
        out_shape=jax.ShapeDtypeStruct((_N, _D), jnp.float32),
    )(x, eigvec, spec2, bo2)
    return out
